# baseline (speedup 1.0000x reference)
import functools

import jax
import jax.numpy as jnp
from jax import lax
from jax.experimental import pallas as pl
from jax.experimental.pallas import tpu as pltpu
from jax.experimental.pallas import tpu_sc as plsc

N = 100000
D = 128
NC = 2
NS = 16
NW = NC * NS
NSEEDS = 15000
SPW = 512
SEEDS_PAD = NW * SPW
CHUNK = 128
NCHUNK = SPW // CHUNK

_BLK = 2000


def _copy_body(emb_ref, out_ref):
    out_ref[...] = emb_ref[...]


def _tc_copy(embeds):
    return pl.pallas_call(
        _copy_body,
        grid=(N // _BLK,),
        in_specs=[pl.BlockSpec((_BLK, D), lambda i: (i, 0))],
        out_specs=pl.BlockSpec((_BLK, D), lambda i: (i, 0)),
        out_shape=jax.ShapeDtypeStruct((N, D), jnp.float32),
    )(embeds)


_mesh = plsc.VectorSubcoreMesh(core_axis_name="c", subcore_axis_name="s")


@functools.partial(
    pl.kernel,
    mesh=_mesh,
    out_type=(),
    scratch_types=[
        pltpu.VMEM((NCHUNK, CHUNK), jnp.int32),
        pltpu.VMEM((CHUNK, D), jnp.float32),
        pltpu.SemaphoreType.DMA,
    ],
)
def _scatter_kernel(out_ref, tok_hbm, seeds_hbm, idx_v, tok_v, sem):
    wid = lax.axis_index("s") * NC + lax.axis_index("c")

    pltpu.sync_copy(seeds_hbm.at[wid], idx_v)

    pltpu.sync_copy(tok_hbm, tok_v)

    copies = [
        pltpu.async_copy(tok_v, out_ref.at[idx_v.at[j]], sem)
        for j in range(NCHUNK)
    ]
    for c in copies:
        c.wait()


RPW = N // NW


@functools.partial(
    pl.kernel,
    mesh=_mesh,
    out_type=jax.ShapeDtypeStruct((NW, RPW, D), jnp.float32),
)
def _sc_copy_probe(emb_hbm, out_hbm):
    wid = lax.axis_index("s") * NC + lax.axis_index("c")
    pltpu.sync_copy(emb_hbm.at[wid], out_hbm.at[wid])


def kernel(embeds, mask_token, seeds):
    seeds_pad = jnp.concatenate(
        [seeds, seeds[: SEEDS_PAD - NSEEDS]]
    ).reshape(NW, NCHUNK, CHUNK)
    tok_rep = jnp.broadcast_to(mask_token, (CHUNK, D))
    out = _sc_copy_probe(embeds.reshape(NW, RPW, D)).reshape(N, D)
    out_ref = jax.new_ref(out)
    _scatter_kernel(out_ref, tok_rep, seeds_pad)
    return (out_ref[...], seeds)


# speedup vs baseline: 25.6931x; 25.6931x over previous
import functools

import jax
import jax.numpy as jnp
from jax import lax
from jax.experimental import pallas as pl
from jax.experimental.pallas import tpu as pltpu
from jax.experimental.pallas import tpu_sc as plsc

N = 100000
D = 128
NC = 2
NS = 16
NW = NC * NS
NSEEDS = 15000
SPW = 512
SEEDS_PAD = NW * SPW
CHUNK = 128
NCHUNK = SPW // CHUNK

_BLK = 4000


def _copy_body(emb_ref, out_ref):
    out_ref[...] = emb_ref[...]


def _tc_copy(embeds):
    return pl.pallas_call(
        _copy_body,
        grid=(N // _BLK,),
        in_specs=[pl.BlockSpec((_BLK, D), lambda i: (i, 0))],
        out_specs=pl.BlockSpec((_BLK, D), lambda i: (i, 0)),
        out_shape=jax.ShapeDtypeStruct((N, D), jnp.float32),
    )(embeds)


_mesh = plsc.VectorSubcoreMesh(core_axis_name="c", subcore_axis_name="s")


@functools.partial(
    pl.kernel,
    mesh=_mesh,
    out_type=(),
    scratch_types=[
        pltpu.VMEM((NCHUNK, CHUNK), jnp.int32),
        pltpu.VMEM((CHUNK, D), jnp.float32),
        pltpu.SemaphoreType.DMA,
    ],
)
def _scatter_kernel(out_ref, tok_hbm, seeds_hbm, idx_v, tok_v, sem):
    wid = lax.axis_index("s") * NC + lax.axis_index("c")

    pltpu.sync_copy(seeds_hbm.at[wid], idx_v)

    pltpu.sync_copy(tok_hbm, tok_v)

    copies = [
        pltpu.async_copy(tok_v, out_ref.at[idx_v.at[j]], sem)
        for j in range(NCHUNK)
    ]
    for c in copies:
        c.wait()


RPW = N // NW


@functools.partial(
    pl.kernel,
    mesh=_mesh,
    out_type=jax.ShapeDtypeStruct((NW, RPW, D), jnp.float32),
)
def _sc_copy_probe(emb_hbm, out_hbm):
    wid = lax.axis_index("s") * NC + lax.axis_index("c")
    pltpu.sync_copy(emb_hbm.at[wid], out_hbm.at[wid])


def kernel(embeds, mask_token, seeds):
    seeds_pad = jnp.concatenate(
        [seeds, seeds[: SEEDS_PAD - NSEEDS]]
    ).reshape(NW, NCHUNK, CHUNK)
    tok_rep = jnp.broadcast_to(mask_token, (CHUNK, D))
    out = _tc_copy(embeds)
    out_ref = jax.new_ref(out)
    _scatter_kernel(out_ref, tok_rep, seeds_pad)
    return (out_ref[...], seeds)


# speedup vs baseline: 27.3429x; 1.0642x over previous
import functools

import jax
import jax.numpy as jnp
from jax import lax
from jax.experimental import pallas as pl
from jax.experimental.pallas import tpu as pltpu
from jax.experimental.pallas import tpu_sc as plsc

N = 100000
D = 128
NC = 2
NS = 16
NW = NC * NS
NSEEDS = 15000
SPW = 512
SEEDS_PAD = NW * SPW
CHUNK = 128
NCHUNK = SPW // CHUNK

_BLK = 10000


def _copy_body(emb_ref, out_ref):
    out_ref[...] = emb_ref[...]


def _tc_copy(embeds):
    return pl.pallas_call(
        _copy_body,
        grid=(N // _BLK,),
        in_specs=[pl.BlockSpec((_BLK, D), lambda i: (i, 0))],
        out_specs=pl.BlockSpec((_BLK, D), lambda i: (i, 0)),
        out_shape=jax.ShapeDtypeStruct((N, D), jnp.float32),
    )(embeds)


_mesh = plsc.VectorSubcoreMesh(core_axis_name="c", subcore_axis_name="s")


@functools.partial(
    pl.kernel,
    mesh=_mesh,
    out_type=(),
    scratch_types=[
        pltpu.VMEM((NCHUNK, CHUNK), jnp.int32),
        pltpu.VMEM((CHUNK, D), jnp.float32),
        pltpu.SemaphoreType.DMA,
    ],
)
def _scatter_kernel(out_ref, tok_hbm, seeds_hbm, idx_v, tok_v, sem):
    wid = lax.axis_index("s") * NC + lax.axis_index("c")

    pltpu.sync_copy(seeds_hbm.at[wid], idx_v)

    pltpu.sync_copy(tok_hbm, tok_v)

    copies = [
        pltpu.async_copy(tok_v, out_ref.at[idx_v.at[j]], sem)
        for j in range(NCHUNK)
    ]
    for c in copies:
        c.wait()


RPW = N // NW


@functools.partial(
    pl.kernel,
    mesh=_mesh,
    out_type=jax.ShapeDtypeStruct((NW, RPW, D), jnp.float32),
)
def _sc_copy_probe(emb_hbm, out_hbm):
    wid = lax.axis_index("s") * NC + lax.axis_index("c")
    pltpu.sync_copy(emb_hbm.at[wid], out_hbm.at[wid])


def kernel(embeds, mask_token, seeds):
    seeds_pad = jnp.concatenate(
        [seeds, seeds[: SEEDS_PAD - NSEEDS]]
    ).reshape(NW, NCHUNK, CHUNK)
    tok_rep = jnp.broadcast_to(mask_token, (CHUNK, D))
    out = _tc_copy(embeds)
    out_ref = jax.new_ref(out)
    _scatter_kernel(out_ref, tok_rep, seeds_pad)
    return (out_ref[...], seeds)


# speedup vs baseline: 27.8894x; 1.0200x over previous
import functools

import jax
import jax.numpy as jnp
from jax import lax
from jax.experimental import pallas as pl
from jax.experimental.pallas import tpu as pltpu
from jax.experimental.pallas import tpu_sc as plsc

N = 100000
D = 128
NC = 2
NS = 16
NW = NC * NS
NSEEDS = 15000
SPW = 512
SEEDS_PAD = NW * SPW
CHUNK = 128
NCHUNK = SPW // CHUNK

_BLK = 20000


def _copy_body(emb_ref, out_ref):
    out_ref[...] = emb_ref[...]


def _tc_copy(embeds):
    return pl.pallas_call(
        _copy_body,
        grid=(N // _BLK,),
        in_specs=[pl.BlockSpec((_BLK, D), lambda i: (i, 0))],
        out_specs=pl.BlockSpec((_BLK, D), lambda i: (i, 0)),
        out_shape=jax.ShapeDtypeStruct((N, D), jnp.float32),
    )(embeds)


_mesh = plsc.VectorSubcoreMesh(core_axis_name="c", subcore_axis_name="s")


@functools.partial(
    pl.kernel,
    mesh=_mesh,
    out_type=(),
    scratch_types=[
        pltpu.VMEM((NCHUNK, CHUNK), jnp.int32),
        pltpu.VMEM((CHUNK, D), jnp.float32),
        pltpu.SemaphoreType.DMA,
    ],
)
def _scatter_kernel(out_ref, tok_hbm, seeds_hbm, idx_v, tok_v, sem):
    wid = lax.axis_index("s") * NC + lax.axis_index("c")

    pltpu.sync_copy(seeds_hbm.at[wid], idx_v)

    pltpu.sync_copy(tok_hbm, tok_v)

    copies = [
        pltpu.async_copy(tok_v, out_ref.at[idx_v.at[j]], sem)
        for j in range(NCHUNK)
    ]
    for c in copies:
        c.wait()


RPW = N // NW


@functools.partial(
    pl.kernel,
    mesh=_mesh,
    out_type=jax.ShapeDtypeStruct((NW, RPW, D), jnp.float32),
)
def _sc_copy_probe(emb_hbm, out_hbm):
    wid = lax.axis_index("s") * NC + lax.axis_index("c")
    pltpu.sync_copy(emb_hbm.at[wid], out_hbm.at[wid])


def kernel(embeds, mask_token, seeds):
    seeds_pad = jnp.concatenate(
        [seeds, seeds[: SEEDS_PAD - NSEEDS]]
    ).reshape(NW, NCHUNK, CHUNK)
    tok_rep = jnp.broadcast_to(mask_token, (CHUNK, D))
    out = _tc_copy(embeds)
    out_ref = jax.new_ref(out)
    _scatter_kernel(out_ref, tok_rep, seeds_pad)
    return (out_ref[...], seeds)
